# baseline (speedup 1.0000x reference)
import functools

import jax
import jax.numpy as jnp
from jax import lax
from jax.experimental import pallas as pl
from jax.experimental.pallas import tpu as pltpu
from jax.experimental.pallas import tpu_sc as plsc

VOCAB = 1000000
H = 64
C = 4
B = 16384
L = 50
N = B * L
NP = N // 2

NC = 2
NS = 16
NW = NC * NS
ROWS_PER_W = NP // NW
PCHUNK = 64
CHUNKS_PER_W = ROWS_PER_W // PCHUNK

BB = 2048
HALF = BB // 2
NCB = B // BB
PPB = HALF

TCB = 1024
TGRID = -(-(VOCAB + 1) // TCB)
VPAD = TGRID * TCB


def _pack_body(tt_ref, out_ref):
    x = tt_ref[...]
    out_ref[...] = jnp.concatenate(
        [x[:, :TCB // 2].T, x[:, TCB // 2:].T], axis=1)


def _tc_pack(tt):
    return pl.pallas_call(
        _pack_body,
        grid=(TGRID,),
        in_specs=[pl.BlockSpec((H, TCB), lambda g: (0, g))],
        out_specs=pl.BlockSpec((TCB // 2, 128), lambda g: (g, 0)),
        out_shape=jax.ShapeDtypeStruct((VPAD // 2, 128), jnp.float32),
    )(tt)


@functools.cache
def _make_sc_gather():
    mesh = plsc.VectorSubcoreMesh(core_axis_name="c", subcore_axis_name="s")

    @functools.partial(
        pl.kernel,
        out_type=jax.ShapeDtypeStruct((NP, 128), jnp.float32),
        mesh=mesh,
        scratch_types=[
            pltpu.VMEM((CHUNKS_PER_W, 2, PCHUNK), jnp.int32),
            pltpu.VMEM((PCHUNK, H), jnp.float32),
            pltpu.VMEM((PCHUNK, H), jnp.float32),
            pltpu.SemaphoreType.DMA,
        ],
        compiler_params=pltpu.CompilerParams(use_tc_tiling_on_sc=False),
    )
    def _sc_gather(ids_hbm, table_hbm, out_hbm, idx_v, lo_v, hi_v, sem):
        wid = lax.axis_index("s") * NC + lax.axis_index("c")
        pltpu.sync_copy(ids_hbm.at[wid], idx_v)
        base = wid * ROWS_PER_W

        def step(j, carry):
            lo = pltpu.async_copy(table_hbm.at[idx_v.at[j, 0]], lo_v, sem)
            hi = pltpu.async_copy(table_hbm.at[idx_v.at[j, 1]], hi_v, sem)
            lo.wait()
            hi.wait()
            rows = pl.ds(base + j * PCHUNK, PCHUNK)
            pltpu.sync_copy(lo_v, out_hbm.at[rows, pl.ds(0, H)])
            pltpu.sync_copy(hi_v, out_hbm.at[rows, pl.ds(H, H)])
            return carry

        lax.fori_loop(0, CHUNKS_PER_W, step, 0)

    return _sc_gather


def _mm_body(gp_ref, ctx_ref, w1_ref, wct_ref, be_ref, out_ref):
    a = gp_ref[...]
    w1 = w1_ref[...]
    dn = (((1,), (1,)), ((), ()))
    r_lo = lax.dot_general(w1, a[:, :H], dn, preferred_element_type=jnp.float32)
    r_hi = lax.dot_general(w1, a[:, H:], dn, preferred_element_type=jnp.float32)
    rc = jnp.dot(wct_ref[...], ctx_ref[0], preferred_element_type=jnp.float32)
    acc = jnp.concatenate([r_lo, r_hi], axis=1) + rc + be_ref[...]
    out_ref[0] = acc


def _tc_fused(gp, ctx_t, w1, wct_t, b_eff):
    return pl.pallas_call(
        _mm_body,
        grid=(L, NCB),
        in_specs=[
            pl.BlockSpec((PPB, 128), lambda l, c: (l * NCB + c, 0)),
            pl.BlockSpec((1, C, BB), lambda l, c: (l, 0, c)),
            pl.BlockSpec((H, H), lambda l, c: (0, 0)),
            pl.BlockSpec((H, C), lambda l, c: (0, 0)),
            pl.BlockSpec((H, 1), lambda l, c: (0, 0)),
        ],
        out_specs=pl.BlockSpec((1, H, BB), lambda l, c: (l, 0, c)),
        out_shape=jax.ShapeDtypeStruct((L, H, B), jnp.float32),
    )(gp, ctx_t, w1, wct_t, b_eff)


def kernel(item_ids, context_features, item_table, W_ctx, b_ctx, W_joint, b_joint):
    idt = item_ids.astype(jnp.int32).T
    g = idt // TCB
    u = idt % TCB
    idt = g * TCB + jnp.where(u < TCB // 2, 2 * u, 2 * (u - TCB // 2) + 1)
    pairs = idt.reshape(L, NCB, 2, HALF)
    pairs = pairs.transpose(0, 1, 3, 2)
    ids4 = pairs.reshape(NW, CHUNKS_PER_W, PCHUNK, 2).transpose(0, 1, 3, 2)

    t_lin = _tc_pack(item_table.T).reshape(VPAD, H)
    gp = _make_sc_gather()(ids4, t_lin)

    W1 = W_joint[:, :H]
    W2 = W_joint[:, H:]
    wct_t = W2 @ W_ctx
    b_eff = (b_joint + W2 @ b_ctx).reshape(H, 1)

    ctx_t = context_features.transpose(1, 2, 0)
    out_t = _tc_fused(gp, ctx_t, W1, wct_t, b_eff)
    return out_t.transpose(2, 0, 1)


# speedup vs baseline: 1.1075x; 1.1075x over previous
import functools

import jax
import jax.numpy as jnp
from jax import lax
from jax.experimental import pallas as pl
from jax.experimental.pallas import tpu as pltpu
from jax.experimental.pallas import tpu_sc as plsc

VOCAB = 1000000
H = 64
C = 4
B = 16384
L = 50
N = B * L
NP = N // 2

NC = 2
NS = 16
NW = NC * NS
ROWS_PER_W = NP // NW
PCHUNK = 64
CHUNKS_PER_W = ROWS_PER_W // PCHUNK

BB = 2048
HALF = BB // 2
NCB = B // BB
PPB = HALF

TCB = 1024
TGRID = -(-(VOCAB + 1) // TCB)
VPAD = TGRID * TCB


def _pack_body(tt_ref, out_ref):
    x = tt_ref[...]
    out_ref[...] = jnp.concatenate(
        [x[:, :TCB // 2].T, x[:, TCB // 2:].T], axis=1)


def _tc_pack(tt):
    return pl.pallas_call(
        _pack_body,
        grid=(TGRID,),
        in_specs=[pl.BlockSpec((H, TCB), lambda g: (0, g))],
        out_specs=pl.BlockSpec((TCB // 2, 128), lambda g: (g, 0)),
        out_shape=jax.ShapeDtypeStruct((VPAD // 2, 128), jnp.float32),
    )(tt)


@functools.cache
def _make_sc_gather(chunks):
    mesh = plsc.VectorSubcoreMesh(core_axis_name="c", subcore_axis_name="s")
    rows_per_w = chunks * PCHUNK

    @functools.partial(
        pl.kernel,
        out_type=jax.ShapeDtypeStruct((NW * rows_per_w, 128), jnp.float32),
        mesh=mesh,
        scratch_types=[
            pltpu.VMEM((chunks, 2, PCHUNK), jnp.int32),
            pltpu.VMEM((PCHUNK, H), jnp.float32),
            pltpu.VMEM((PCHUNK, H), jnp.float32),
            pltpu.SemaphoreType.DMA,
        ],
        compiler_params=pltpu.CompilerParams(use_tc_tiling_on_sc=False),
    )
    def _sc_gather(ids_hbm, table_hbm, out_hbm, idx_v, lo_v, hi_v, sem):
        wid = lax.axis_index("s") * NC + lax.axis_index("c")
        pltpu.sync_copy(ids_hbm.at[wid], idx_v)
        base = wid * rows_per_w

        def step(j, carry):
            lo = pltpu.async_copy(table_hbm.at[idx_v.at[j, 0]], lo_v, sem)
            hi = pltpu.async_copy(table_hbm.at[idx_v.at[j, 1]], hi_v, sem)
            lo.wait()
            hi.wait()
            rows = pl.ds(base + j * PCHUNK, PCHUNK)
            pltpu.sync_copy(lo_v, out_hbm.at[rows, pl.ds(0, H)])
            pltpu.sync_copy(hi_v, out_hbm.at[rows, pl.ds(H, H)])
            return carry

        lax.fori_loop(0, chunks, step, 0)

    return _sc_gather


def _mm_body(gp_ref, ctx_ref, w1_ref, wct_ref, be_ref, out_ref):
    a = gp_ref[...]
    w1 = w1_ref[...]
    dn = (((1,), (1,)), ((), ()))
    r_lo = lax.dot_general(w1, a[:, :H], dn, preferred_element_type=jnp.float32)
    r_hi = lax.dot_general(w1, a[:, H:], dn, preferred_element_type=jnp.float32)
    rc = jnp.dot(wct_ref[...], ctx_ref[0], preferred_element_type=jnp.float32)
    acc = jnp.concatenate([r_lo, r_hi], axis=1) + rc + be_ref[...]
    out_ref[0] = acc


def _mm_body_acc(gp_ref, ctx_ref, w1_ref, wct_ref, be_ref, prev_ref, out_ref):
    del prev_ref
    _mm_body(gp_ref, ctx_ref, w1_ref, wct_ref, be_ref, out_ref)


def _tc_fused_part(gp, ctx_t, w1, wct_t, b_eff, l0, lcount, prev):
    common = dict(
        grid=(lcount, NCB),
        out_specs=pl.BlockSpec((1, H, BB), lambda l, c: (l + l0, 0, c)),
        out_shape=jax.ShapeDtypeStruct((L, H, B), jnp.float32),
    )
    in_specs = [
        pl.BlockSpec((PPB, 128), lambda l, c: (l * NCB + c, 0)),
        pl.BlockSpec((1, C, BB), lambda l, c: (l + l0, 0, c)),
        pl.BlockSpec((H, H), lambda l, c: (0, 0)),
        pl.BlockSpec((H, C), lambda l, c: (0, 0)),
        pl.BlockSpec((H, 1), lambda l, c: (0, 0)),
    ]
    if prev is None:
        return pl.pallas_call(_mm_body, in_specs=in_specs, **common)(
            gp, ctx_t, w1, wct_t, b_eff)
    return pl.pallas_call(
        _mm_body_acc,
        in_specs=in_specs + [pl.BlockSpec(memory_space=pl.ANY)],
        input_output_aliases={5: 0},
        **common,
    )(gp, ctx_t, w1, wct_t, b_eff, prev)


def kernel(item_ids, context_features, item_table, W_ctx, b_ctx, W_joint, b_joint):
    idt = item_ids.astype(jnp.int32).T
    g = idt // TCB
    u = idt % TCB
    idt = g * TCB + jnp.where(u < TCB // 2, 2 * u, 2 * (u - TCB // 2) + 1)
    pairs = idt.reshape(L, NCB, 2, HALF)
    pairs = pairs.transpose(0, 1, 3, 2)

    t_lin = _tc_pack(item_table.T).reshape(VPAD, H)

    W1 = W_joint[:, :H]
    W2 = W_joint[:, H:]
    wct_t = W2 @ W_ctx
    b_eff = (b_joint + W2 @ b_ctx).reshape(H, 1)
    ctx_t = context_features.transpose(1, 2, 0)

    S = 2
    LS = L // S
    chunks = LS * NCB * HALF // (NW * PCHUNK)
    sc_gather = _make_sc_gather(chunks)
    gps = []
    for s in range(S):
        ids4 = (pairs[s * LS:(s + 1) * LS]
                .reshape(NW, chunks, PCHUNK, 2).transpose(0, 1, 3, 2))
        gps.append(sc_gather(ids4, t_lin))
    out_t = None
    for s in range(S):
        out_t = _tc_fused_part(gps[s], ctx_t, W1, wct_t, b_eff,
                               s * LS, LS, out_t)
    return out_t.transpose(2, 0, 1)
